# baseline (speedup 1.0000x reference)
import jax
import jax.numpy as jnp
from jax import lax
from jax.experimental import pallas as pl
from jax.experimental.pallas import tpu as pltpu
from jax.experimental.pallas import tpu_sc as plsc

N_NODES = 10000
N_EDGES = 320000
D_IN = 128
DIM = 32
BN_EPS = 1e-5

NC = 2
NS = 16
N_PAD = 10240
ROWS_PER_SUB = N_PAD // NS


def _make_col_body(nbuf, chunk, nchunk, dimh):
  def _body(y_hbm, src_hbm, dst_hbm, zeros_hbm, out_hbm,
            src_v, dst_v, rows_v, acc_sh, *sems):
    c = lax.axis_index("c")
    s = lax.axis_index("s")

    pltpu.sync_copy(src_hbm.at[c, s], src_v)
    pltpu.sync_copy(dst_hbm.at[s], dst_v)
    for b in range(nbuf):
        pltpu.async_copy(y_hbm.at[src_v.at[b]], rows_v.at[b], sems[b])
    pltpu.sync_copy(zeros_hbm.at[pl.ds(s * ROWS_PER_SUB, ROWS_PER_SUB)],
                    acc_sh.at[pl.ds(s * ROWS_PER_SUB, ROWS_PER_SUB)])
    plsc.subcore_barrier()

    def body(g, carry):
        for b in range(nbuf):
            j = g * nbuf + b
            pltpu.make_async_copy(y_hbm.at[src_v.at[j]], rows_v.at[b],
                                  sems[b]).wait()
            pltpu.sync_copy(rows_v.at[b], acc_sh.at[dst_v.at[j]], add=True)

            @pl.when(j + nbuf < nchunk)
            def _():
                pltpu.async_copy(y_hbm.at[src_v.at[j + nbuf]],
                                 rows_v.at[b], sems[b])
        return carry

    lax.fori_loop(0, nchunk // nbuf, body, 0, unroll=False)

    plsc.subcore_barrier()
    pltpu.sync_copy(acc_sh.at[pl.ds(s * ROWS_PER_SUB, ROWS_PER_SUB)],
                    out_hbm.at[pl.ds(s * ROWS_PER_SUB, ROWS_PER_SUB),
                               pl.ds(c * dimh, dimh)])
  return _body


def _make_edge_body(nbuf, chunk, nchunk):
  def _body(y_hbm, src_hbm, dst_hbm, zeros_hbm, out_hbm,
            src_v, dst_v, rows_v, acc_sh, *sems):
    c = lax.axis_index("c")
    s = lax.axis_index("s")
    wid = s * NC + c

    pltpu.sync_copy(src_hbm.at[wid], src_v)
    pltpu.sync_copy(dst_hbm.at[wid], dst_v)
    for b in range(nbuf):
        pltpu.async_copy(y_hbm.at[src_v.at[b]], rows_v.at[b], sems[b])
    pltpu.sync_copy(zeros_hbm.at[pl.ds(s * ROWS_PER_SUB, ROWS_PER_SUB)],
                    acc_sh.at[pl.ds(s * ROWS_PER_SUB, ROWS_PER_SUB)])
    plsc.subcore_barrier()

    def body(g, carry):
        for b in range(nbuf):
            j = g * nbuf + b
            pltpu.make_async_copy(y_hbm.at[src_v.at[j]], rows_v.at[b],
                                  sems[b]).wait()
            pltpu.sync_copy(rows_v.at[b], acc_sh.at[dst_v.at[j]], add=True)

            @pl.when(j + nbuf < nchunk)
            def _():
                pltpu.async_copy(y_hbm.at[src_v.at[j + nbuf]], rows_v.at[b],
                                 sems[b])
        return carry

    lax.fori_loop(0, nchunk // nbuf, body, 0, unroll=False)

    plsc.subcore_barrier()
    pltpu.sync_copy(acc_sh.at[pl.ds(s * ROWS_PER_SUB, ROWS_PER_SUB)],
                    out_hbm.at[c, pl.ds(s * ROWS_PER_SUB, ROWS_PER_SUB)])
  return _body


def _seg_sum_cols(y, src, dst, zeros, dim, nbuf, chunk):
    dimh = dim // 2
    nchunk = (N_EDGES // NS) // chunk
    yv = y.reshape(2 * N_NODES, dimh)
    st = src.reshape(NS, nchunk, chunk)
    src2 = jnp.stack([2 * st, 2 * st + 1])
    mesh = plsc.VectorSubcoreMesh(core_axis_name="c", subcore_axis_name="s")
    fn = pl.kernel(
        _make_col_body(nbuf, chunk, nchunk, dimh),
        out_type=jax.ShapeDtypeStruct((N_PAD, dim), jnp.float32),
        mesh=mesh,
        compiler_params=pltpu.CompilerParams(use_tc_tiling_on_sc=False),
        scratch_types=[
            pltpu.VMEM((nchunk, chunk), jnp.int32),
            pltpu.VMEM((nchunk, chunk), jnp.int32),
            pltpu.VMEM((nbuf, chunk, dimh), jnp.float32),
            pltpu.VMEM_SHARED((N_PAD, dimh), jnp.float32),
        ] + [pltpu.SemaphoreType.DMA] * nbuf,
    )
    return fn(yv, src2, dst.reshape(NS, nchunk, chunk), zeros)


def _seg_sum_edges(y, src, dst, zeros, dim, nbuf, chunk):
    nchunk = (N_EDGES // (NC * NS)) // chunk
    mesh = plsc.VectorSubcoreMesh(core_axis_name="c", subcore_axis_name="s")
    fn = pl.kernel(
        _make_edge_body(nbuf, chunk, nchunk),
        out_type=jax.ShapeDtypeStruct((NC, N_PAD, dim), jnp.float32),
        mesh=mesh,
        compiler_params=pltpu.CompilerParams(use_tc_tiling_on_sc=False),
        scratch_types=[
            pltpu.VMEM((nchunk, chunk), jnp.int32),
            pltpu.VMEM((nchunk, chunk), jnp.int32),
            pltpu.VMEM((nbuf, chunk, dim), jnp.float32),
            pltpu.VMEM_SHARED((N_PAD, dim), jnp.float32),
        ] + [pltpu.SemaphoreType.DMA] * nbuf,
    )
    return fn(y, src.reshape(NC * NS, nchunk, chunk),
              dst.reshape(NC * NS, nchunk, chunk), zeros)


def _bn(h, g, be):
    mu = jnp.mean(h, axis=0, keepdims=True)
    hc = h - mu
    var = jnp.mean(hc * hc, axis=0, keepdims=True)
    return hc * lax.rsqrt(var + BN_EPS) * g + be


def _layer1_body(x_ref, agg_ref, wa_ref, ba_ref, wb_ref, bb_ref, g_ref,
                 be_ref, o_ref):
    h0 = x_ref[...] + agg_ref[:N_NODES]
    t = jnp.maximum(
        jnp.dot(h0, wa_ref[...], preferred_element_type=jnp.float32)
        + ba_ref[...], 0.0)
    h = jnp.dot(t, wb_ref[...], preferred_element_type=jnp.float32) \
        + bb_ref[...]
    o_ref[...] = _bn(jnp.maximum(h, 0.0), g_ref[...], be_ref[...])


def _layer2_body(h_ref, agg_ref, wa_ref, ba_ref, wb_ref, bb_ref, g_ref,
                 be_ref, o_ref):
    h2 = h_ref[...] + agg_ref[0, :N_NODES] + agg_ref[1, :N_NODES]
    t = jnp.maximum(
        jnp.dot(h2, wa_ref[...], preferred_element_type=jnp.float32)
        + ba_ref[...], 0.0)
    h = jnp.dot(t, wb_ref[...], preferred_element_type=jnp.float32) \
        + bb_ref[...]
    o_ref[...] = _bn(jnp.maximum(h, 0.0), g_ref[...], be_ref[...])


def _layer1_tc(x, agg, wa, ba, wb, bb, g, be):
    return pl.pallas_call(
        _layer1_body,
        out_shape=jax.ShapeDtypeStruct((N_NODES, DIM), jnp.float32),
    )(x, agg, wa, ba, wb, bb, g, be)


def _layer2_tc(h, agg, wa, ba, wb, bb, g, be):
    return pl.pallas_call(
        _layer2_body,
        out_shape=jax.ShapeDtypeStruct((N_NODES, D_IN), jnp.float32),
    )(h, agg, wa, ba, wb, bb, g, be)


def kernel(x, edge_index, W1a, b1a, W1b, b1b, g1, be1,
           W2a, b2a, W2b, b2b, g2, be2):
    ei = edge_index.astype(jnp.int32)
    src = ei[0]
    dst = ei[1]
    zeros64 = jnp.zeros((N_PAD, D_IN // 2), jnp.float32)
    zeros32 = jnp.zeros((N_PAD, DIM), jnp.float32)

    p1 = _seg_sum_cols(x, src, dst, zeros64, D_IN, 5, 125)
    h1 = _layer1_tc(x, p1, W1a, b1a.reshape(1, DIM), W1b,
                    b1b.reshape(1, DIM), g1.reshape(1, DIM),
                    be1.reshape(1, DIM))
    p2 = _seg_sum_edges(h1, src, dst, zeros32, DIM, 16, 125)
    out = _layer2_tc(h1, p2, W2a, b2a.reshape(1, DIM), W2b,
                     b2b.reshape(1, D_IN), g2.reshape(1, D_IN),
                     be2.reshape(1, D_IN))
    return out


# speedup vs baseline: 1.0054x; 1.0054x over previous
import jax
import jax.numpy as jnp
from jax import lax
from jax.experimental import pallas as pl
from jax.experimental.pallas import tpu as pltpu
from jax.experimental.pallas import tpu_sc as plsc

N_NODES = 10000
N_EDGES = 320000
D_IN = 128
DIM = 32
BN_EPS = 1e-5

NC = 2
NS = 16
N_PAD = 10240
ROWS_PER_SUB = N_PAD // NS


def _make_col_body(nbuf, chunk, nchunk, dimh):
  def _body(y_hbm, src_hbm, dst_hbm, zeros_hbm, out_hbm,
            src_v, dst_v, rows_v, acc_sh, *sems):
    c = lax.axis_index("c")
    s = lax.axis_index("s")

    pltpu.sync_copy(src_hbm.at[c, s], src_v)
    pltpu.sync_copy(dst_hbm.at[s], dst_v)
    for b in range(nbuf):
        pltpu.async_copy(y_hbm.at[src_v.at[b]], rows_v.at[b], sems[b])
    pltpu.sync_copy(zeros_hbm.at[pl.ds(s * ROWS_PER_SUB, ROWS_PER_SUB)],
                    acc_sh.at[pl.ds(s * ROWS_PER_SUB, ROWS_PER_SUB)])
    plsc.subcore_barrier()

    def body(g, carry):
        for b in range(nbuf):
            j = g * nbuf + b
            pltpu.make_async_copy(y_hbm.at[src_v.at[j]], rows_v.at[b],
                                  sems[b]).wait()
            pltpu.sync_copy(rows_v.at[b], acc_sh.at[dst_v.at[j]], add=True)

            @pl.when(j + nbuf < nchunk)
            def _():
                pltpu.async_copy(y_hbm.at[src_v.at[j + nbuf]],
                                 rows_v.at[b], sems[b])
        return carry

    lax.fori_loop(0, nchunk // nbuf, body, 0, unroll=False)

    plsc.subcore_barrier()
    pltpu.sync_copy(acc_sh.at[pl.ds(s * ROWS_PER_SUB, ROWS_PER_SUB)],
                    out_hbm.at[pl.ds(s * ROWS_PER_SUB, ROWS_PER_SUB),
                               pl.ds(c * dimh, dimh)])
  return _body


def _make_edge_body(nbuf, chunk, nchunk):
  def _body(y_hbm, src_hbm, dst_hbm, zeros_hbm, out_hbm,
            src_v, dst_v, rows_v, acc_sh, *sems):
    c = lax.axis_index("c")
    s = lax.axis_index("s")
    wid = s * NC + c

    pltpu.sync_copy(src_hbm.at[wid], src_v)
    pltpu.sync_copy(dst_hbm.at[wid], dst_v)
    for b in range(nbuf):
        pltpu.async_copy(y_hbm.at[src_v.at[b]], rows_v.at[b], sems[b])
    pltpu.sync_copy(zeros_hbm.at[pl.ds(s * ROWS_PER_SUB, ROWS_PER_SUB)],
                    acc_sh.at[pl.ds(s * ROWS_PER_SUB, ROWS_PER_SUB)])
    plsc.subcore_barrier()

    def body(g, carry):
        for b in range(nbuf):
            j = g * nbuf + b
            pltpu.make_async_copy(y_hbm.at[src_v.at[j]], rows_v.at[b],
                                  sems[b]).wait()
            pltpu.sync_copy(rows_v.at[b], acc_sh.at[dst_v.at[j]], add=True)

            @pl.when(j + nbuf < nchunk)
            def _():
                pltpu.async_copy(y_hbm.at[src_v.at[j + nbuf]], rows_v.at[b],
                                 sems[b])
        return carry

    lax.fori_loop(0, nchunk // nbuf, body, 0, unroll=False)

    plsc.subcore_barrier()
    pltpu.sync_copy(acc_sh.at[pl.ds(s * ROWS_PER_SUB, ROWS_PER_SUB)],
                    out_hbm.at[c, pl.ds(s * ROWS_PER_SUB, ROWS_PER_SUB)])
  return _body


def _seg_sum_cols(y, src, dst, zeros, dim, nbuf, chunk):
    dimh = dim // 2
    nchunk = (N_EDGES // NS) // chunk
    yv = y.reshape(2 * N_NODES, dimh)
    st = src.reshape(NS, nchunk, chunk)
    src2 = jnp.stack([2 * st, 2 * st + 1])
    mesh = plsc.VectorSubcoreMesh(core_axis_name="c", subcore_axis_name="s")
    fn = pl.kernel(
        _make_col_body(nbuf, chunk, nchunk, dimh),
        out_type=jax.ShapeDtypeStruct((N_PAD, dim), jnp.float32),
        mesh=mesh,
        compiler_params=pltpu.CompilerParams(use_tc_tiling_on_sc=False),
        scratch_types=[
            pltpu.VMEM((nchunk, chunk), jnp.int32),
            pltpu.VMEM((nchunk, chunk), jnp.int32),
            pltpu.VMEM((nbuf, chunk, dimh), jnp.float32),
            pltpu.VMEM_SHARED((N_PAD, dimh), jnp.float32),
        ] + [pltpu.SemaphoreType.DMA] * nbuf,
    )
    return fn(yv, src2, dst.reshape(NS, nchunk, chunk), zeros)


def _seg_sum_edges(y, src, dst, zeros, dim, nbuf, chunk):
    nchunk = (N_EDGES // (NC * NS)) // chunk
    mesh = plsc.VectorSubcoreMesh(core_axis_name="c", subcore_axis_name="s")
    fn = pl.kernel(
        _make_edge_body(nbuf, chunk, nchunk),
        out_type=jax.ShapeDtypeStruct((NC, N_PAD, dim), jnp.float32),
        mesh=mesh,
        compiler_params=pltpu.CompilerParams(use_tc_tiling_on_sc=False),
        scratch_types=[
            pltpu.VMEM((nchunk, chunk), jnp.int32),
            pltpu.VMEM((nchunk, chunk), jnp.int32),
            pltpu.VMEM((nbuf, chunk, dim), jnp.float32),
            pltpu.VMEM_SHARED((N_PAD, dim), jnp.float32),
        ] + [pltpu.SemaphoreType.DMA] * nbuf,
    )
    return fn(y, src.reshape(NC * NS, nchunk, chunk),
              dst.reshape(NC * NS, nchunk, chunk), zeros)


def _bn(h, g, be):
    mu = jnp.mean(h, axis=0, keepdims=True)
    hc = h - mu
    var = jnp.mean(hc * hc, axis=0, keepdims=True)
    return hc * lax.rsqrt(var + BN_EPS) * g + be


def _layer1_body(x_ref, agg_ref, wa_ref, ba_ref, wb_ref, bb_ref, g_ref,
                 be_ref, o_ref):
    h0 = x_ref[...] + agg_ref[:N_NODES]
    t = jnp.maximum(
        jnp.dot(h0, wa_ref[...], preferred_element_type=jnp.float32)
        + ba_ref[...], 0.0)
    h = jnp.dot(t, wb_ref[...], preferred_element_type=jnp.float32) \
        + bb_ref[...]
    o_ref[...] = _bn(jnp.maximum(h, 0.0), g_ref[...], be_ref[...])


def _layer2_body(h_ref, agg_ref, wa_ref, ba_ref, wb_ref, bb_ref, g_ref,
                 be_ref, o_ref):
    h2 = h_ref[...] + agg_ref[0, :N_NODES] + agg_ref[1, :N_NODES]
    t = jnp.maximum(
        jnp.dot(h2, wa_ref[...], preferred_element_type=jnp.float32)
        + ba_ref[...], 0.0)
    h = jnp.dot(t, wb_ref[...], preferred_element_type=jnp.float32) \
        + bb_ref[...]
    o_ref[...] = _bn(jnp.maximum(h, 0.0), g_ref[...], be_ref[...])


def _layer1_tc(x, agg, wa, ba, wb, bb, g, be):
    return pl.pallas_call(
        _layer1_body,
        out_shape=jax.ShapeDtypeStruct((N_NODES, DIM), jnp.float32),
    )(x, agg, wa, ba, wb, bb, g, be)


def _layer2_tc(h, agg, wa, ba, wb, bb, g, be):
    return pl.pallas_call(
        _layer2_body,
        out_shape=jax.ShapeDtypeStruct((N_NODES, D_IN), jnp.float32),
    )(h, agg, wa, ba, wb, bb, g, be)


def kernel(x, edge_index, W1a, b1a, W1b, b1b, g1, be1,
           W2a, b2a, W2b, b2b, g2, be2):
    ei = edge_index.astype(jnp.int32)
    src = ei[0]
    dst = ei[1]
    zeros64 = jnp.zeros((N_PAD, D_IN // 2), jnp.float32)
    zeros32 = jnp.zeros((N_PAD, DIM), jnp.float32)

    p1 = _seg_sum_cols(x, src, dst, zeros64, D_IN, 4, 125)
    h1 = _layer1_tc(x, p1, W1a, b1a.reshape(1, DIM), W1b,
                    b1b.reshape(1, DIM), g1.reshape(1, DIM),
                    be1.reshape(1, DIM))
    p2 = _seg_sum_edges(h1, src, dst, zeros32, DIM, 8, 125)
    out = _layer2_tc(h1, p2, W2a, b2a.reshape(1, DIM), W2b,
                     b2b.reshape(1, D_IN), g2.reshape(1, D_IN),
                     be2.reshape(1, D_IN))
    return out
